# baseline (speedup 1.0000x reference)
import functools

import jax
import jax.numpy as jnp
from jax import lax
from jax.experimental import pallas as pl
from jax.experimental.pallas import tpu as pltpu
from jax.experimental.pallas import tpu_sc as plsc

_NC = 2
_NS = 16
_NW = _NC * _NS

_BT = 256
_CH = 128


def _scores_argmax_body(hr_ref, hi_ref, w1_ref, w2_ref, w3n_ref, idx_ref):
    hr = hr_ref[...]
    hi = hi_ref[...]
    d = hr.shape[0] * hr.shape[1]
    hrt = hr.reshape(d, hr.shape[2])
    hit = hi.reshape(d, hi.shape[2])
    pa = jnp.dot(w1_ref[...], hrt + hit, preferred_element_type=jnp.float32)
    pb = jnp.dot(w2_ref[...], hit, preferred_element_type=jnp.float32)
    pc = jnp.dot(w3n_ref[...], hrt, preferred_element_type=jnp.float32)
    re = pa - pb
    im = pa + pc
    s = re * re + im * im
    idx_ref[0, 0, :] = jnp.argmax(s, axis=0).astype(jnp.int32)


def _tc_scores_argmax(hrt, hit, w1, w2, w3n, nb, tile0):
    s_dim, v_dim, b = hrt.shape
    k_codes = w1.shape[0]
    d = s_dim * v_dim
    wspec = pl.BlockSpec((k_codes, d), lambda i: (0, 0))
    out = pl.pallas_call(
        _scores_argmax_body,
        grid=(nb,),
        in_specs=[
            pl.BlockSpec((s_dim, v_dim, _BT), lambda i: (0, 0, i + tile0)),
            pl.BlockSpec((s_dim, v_dim, _BT), lambda i: (0, 0, i + tile0)),
            wspec, wspec, wspec,
        ],
        out_specs=pl.BlockSpec((1, 1, _BT), lambda i: (i, 0, 0)),
        out_shape=jax.ShapeDtypeStruct((nb, 1, _BT), jnp.int32),
    )(hrt, hit, w1, w2, w3n)
    return out.reshape(nb * _BT)


def _sc_gather(table, idx2, b, d_pad, d_out):
    rows_per_w = b // _NW
    chunks = rows_per_w // _CH
    mesh = plsc.VectorSubcoreMesh(core_axis_name="c", subcore_axis_name="s")
    n16 = d_out // 16

    @functools.partial(
        pl.kernel,
        mesh=mesh,
        out_type=jax.ShapeDtypeStruct((b * d_out,), jnp.float32),
        scratch_types=[
            pltpu.VMEM((chunks, _CH), jnp.int32),
            pltpu.VMEM((2, _CH, d_pad), jnp.float32),
            pltpu.VMEM((_CH * d_out,), jnp.float32),
            pltpu.SemaphoreType.DMA,
        ],
    )
    def gather_kernel(table_hbm, idx_hbm, out_hbm, idx_v, rows_v, packed_v,
                      gsem):
        wid = lax.axis_index("s") * _NC + lax.axis_index("c")
        base = wid * rows_per_w
        pltpu.sync_copy(idx_hbm.at[pl.ds(wid * chunks, chunks)], idx_v)
        copies = [None] * chunks
        copies[0] = pltpu.async_copy(
            table_hbm.at[idx_v.at[0]], rows_v.at[0], gsem)
        for c in range(chunks):
            if c + 1 < chunks:
                copies[c + 1] = pltpu.async_copy(
                    table_hbm.at[idx_v.at[c + 1]],
                    rows_v.at[(c + 1) % 2], gsem)
            copies[c].wait()
            buf = rows_v.at[c % 2]

            def repack_row(r, _):
                for j in range(n16):
                    packed_v[pl.ds(r * d_out + j * 16, 16)] = (
                        buf[r, pl.ds(j * 16, 16)])
                return 0

            lax.fori_loop(0, _CH, repack_row, 0, unroll=False)
            pltpu.sync_copy(
                packed_v,
                out_hbm.at[pl.ds((base + c * _CH) * d_out, _CH * d_out)])

    return gather_kernel(table, idx2)


def kernel(h_est_real, h_est_imag, codebook_real, codebook_imag):
    b, v, s = h_est_real.shape
    k_codes = codebook_real.shape[0]
    d = v * s

    hrt = h_est_real.transpose(2, 1, 0)
    hit = h_est_imag.transpose(2, 1, 0)
    crf = codebook_real.transpose(0, 2, 1).reshape(k_codes, d)
    cif = codebook_imag.transpose(0, 2, 1).reshape(k_codes, d)
    w1 = crf
    w2 = crf - cif
    w3n = -(crf + cif)

    d_pad = 2 * _CH
    table = jnp.stack([codebook_real, codebook_imag], axis=-1)
    table = table.reshape(k_codes, 2 * d)
    table = jnp.pad(table, ((0, 0), (0, d_pad - 2 * d)))

    nb = b // _BT
    idx = _tc_scores_argmax(hrt, hit, w1, w2, w3n, nb, 0)
    idx2 = idx.reshape(b // _CH, _CH)
    flat = _sc_gather(table, idx2, b, d_pad, 2 * d)
    return flat.reshape(b, v, s, 2)


# speedup vs baseline: 7.0688x; 7.0688x over previous
import functools

import jax
import jax.numpy as jnp
from jax import lax
from jax.experimental import pallas as pl
from jax.experimental.pallas import tpu as pltpu
from jax.experimental.pallas import tpu_sc as plsc

_NC = 2
_NS = 16
_NW = _NC * _NS

_BT = 256
_CH = 128


def _scores_argmax_body(hr_ref, hi_ref, w1_ref, w2_ref, w3n_ref, idx_ref):
    hr = hr_ref[...]
    hi = hi_ref[...]
    d = hr.shape[0] * hr.shape[1]
    hrt = hr.reshape(d, hr.shape[2])
    hit = hi.reshape(d, hi.shape[2])
    pa = jnp.dot(w1_ref[...], hrt + hit, preferred_element_type=jnp.float32)
    pb = jnp.dot(w2_ref[...], hit, preferred_element_type=jnp.float32)
    pc = jnp.dot(w3n_ref[...], hrt, preferred_element_type=jnp.float32)
    re = pa - pb
    im = pa + pc
    s = re * re + im * im
    idx_ref[0, 0, :] = jnp.argmax(s, axis=0).astype(jnp.int32)


def _tc_scores_argmax(hrt, hit, w1, w2, w3n, nb, tile0):
    s_dim, v_dim, b = hrt.shape
    k_codes = w1.shape[0]
    d = s_dim * v_dim
    wspec = pl.BlockSpec((k_codes, d), lambda i: (0, 0))
    out = pl.pallas_call(
        _scores_argmax_body,
        grid=(nb,),
        in_specs=[
            pl.BlockSpec((s_dim, v_dim, _BT), lambda i: (0, 0, i + tile0)),
            pl.BlockSpec((s_dim, v_dim, _BT), lambda i: (0, 0, i + tile0)),
            wspec, wspec, wspec,
        ],
        out_specs=pl.BlockSpec((1, 1, _BT), lambda i: (i, 0, 0)),
        out_shape=jax.ShapeDtypeStruct((nb, 1, _BT), jnp.int32),
    )(hrt, hit, w1, w2, w3n)
    return out.reshape(nb * _BT)


def _sc_gather(table, idx2, b, d_pad, d_out):
    rows_per_w = b // _NW
    chunks = rows_per_w // _CH
    mesh = plsc.VectorSubcoreMesh(core_axis_name="c", subcore_axis_name="s")

    @functools.partial(
        pl.kernel,
        mesh=mesh,
        out_type=jax.ShapeDtypeStruct((b, d_pad), jnp.float32),
        scratch_types=[
            pltpu.VMEM((chunks, _CH), jnp.int32),
            pltpu.VMEM((3, _CH, d_pad), jnp.float32),
            pltpu.SemaphoreType.DMA,
            pltpu.SemaphoreType.DMA,
        ],
    )
    def gather_kernel(table_hbm, idx_hbm, out_hbm, idx_v, rows_v, gsem, osem):
        wid = lax.axis_index("s") * _NC + lax.axis_index("c")
        base = wid * rows_per_w
        pltpu.sync_copy(idx_hbm.at[pl.ds(wid * chunks, chunks)], idx_v)
        gathers = [None] * chunks
        outs = [None] * chunks
        waited = set()
        gathers[0] = pltpu.async_copy(
            table_hbm.at[idx_v.at[0]], rows_v.at[0], gsem)
        for c in range(chunks):
            if c + 1 < chunks:
                if c + 1 >= 3:
                    outs[c - 2].wait()
                    waited.add(c - 2)
                gathers[c + 1] = pltpu.async_copy(
                    table_hbm.at[idx_v.at[c + 1]],
                    rows_v.at[(c + 1) % 3], gsem)
            gathers[c].wait()
            outs[c] = pltpu.async_copy(
                rows_v.at[c % 3], out_hbm.at[pl.ds(base + c * _CH, _CH)],
                osem)
        for c in range(chunks):
            if c not in waited:
                outs[c].wait()

    return gather_kernel(table, idx2)


def kernel(h_est_real, h_est_imag, codebook_real, codebook_imag):
    b, v, s = h_est_real.shape
    k_codes = codebook_real.shape[0]
    d = v * s

    hrt = h_est_real.transpose(2, 1, 0)
    hit = h_est_imag.transpose(2, 1, 0)
    crf = codebook_real.transpose(0, 2, 1).reshape(k_codes, d)
    cif = codebook_imag.transpose(0, 2, 1).reshape(k_codes, d)
    w1 = crf
    w2 = crf - cif
    w3n = -(crf + cif)

    d_pad = 2 * _CH
    table = jnp.stack([codebook_real, codebook_imag], axis=-1)
    table = table.reshape(k_codes, 2 * d)
    table = jnp.pad(table, ((0, 0), (0, d_pad - 2 * d)))

    nb = b // _BT
    idx = _tc_scores_argmax(hrt, hit, w1, w2, w3n, nb, 0)
    idx2 = idx.reshape(b // _CH, _CH)
    rows = _sc_gather(table, idx2, b, d_pad, 2 * d)
    return rows[:, : 2 * d].reshape(b, v, s, 2)


# speedup vs baseline: 7.3208x; 1.0357x over previous
import functools

import jax
import jax.numpy as jnp
from jax import lax
from jax.experimental import pallas as pl
from jax.experimental.pallas import tpu as pltpu
from jax.experimental.pallas import tpu_sc as plsc

_NC = 2
_NS = 16
_NW = _NC * _NS

_BT = 256
_CH = 128


def _scores_argmax_body(hr_ref, hi_ref, wre_ref, wim_ref, idx_ref):
    hr = hr_ref[...]
    hi = hi_ref[...]
    d = hr.shape[0] * hr.shape[1]
    hrt = hr.reshape(d, hr.shape[2])
    hit = hi.reshape(d, hi.shape[2])
    xre = jnp.concatenate([hrt + hit, hit], axis=0)
    xim = jnp.concatenate([hrt + hit, hrt], axis=0)
    re = jnp.dot(wre_ref[...], xre, preferred_element_type=jnp.float32)
    im = jnp.dot(wim_ref[...], xim, preferred_element_type=jnp.float32)
    s = re * re + im * im
    idx_ref[0, 0, :] = jnp.argmax(s, axis=0).astype(jnp.int32)


def _tc_scores_argmax(hrt, hit, w1, w2, nb, tile0):
    s_dim, v_dim, b = hrt.shape
    k_codes = w1.shape[0]
    d = s_dim * v_dim
    wspec = pl.BlockSpec((k_codes, 2 * d), lambda i: (0, 0))
    out = pl.pallas_call(
        _scores_argmax_body,
        grid=(nb,),
        in_specs=[
            pl.BlockSpec((s_dim, v_dim, _BT), lambda i: (0, 0, i + tile0)),
            pl.BlockSpec((s_dim, v_dim, _BT), lambda i: (0, 0, i + tile0)),
            wspec, wspec,
        ],
        out_specs=pl.BlockSpec((1, 1, _BT), lambda i: (i, 0, 0)),
        out_shape=jax.ShapeDtypeStruct((nb, 1, _BT), jnp.int32),
    )(hrt, hit, w1, w2)
    return out.reshape(nb * _BT)


def _sc_gather(table, idx2, b, d_pad, d_out):
    rows_per_w = b // _NW
    chunks = rows_per_w // _CH
    mesh = plsc.VectorSubcoreMesh(core_axis_name="c", subcore_axis_name="s")

    @functools.partial(
        pl.kernel,
        mesh=mesh,
        out_type=jax.ShapeDtypeStruct((b, d_pad), jnp.float32),
        scratch_types=[
            pltpu.VMEM((chunks, _CH), jnp.int32),
            pltpu.VMEM((3, _CH, d_pad), jnp.float32),
            pltpu.SemaphoreType.DMA,
            pltpu.SemaphoreType.DMA,
        ],
    )
    def gather_kernel(table_hbm, idx_hbm, out_hbm, idx_v, rows_v, gsem, osem):
        wid = lax.axis_index("s") * _NC + lax.axis_index("c")
        base = wid * rows_per_w
        pltpu.sync_copy(idx_hbm.at[pl.ds(wid * chunks, chunks)], idx_v)
        gathers = [None] * chunks
        outs = [None] * chunks
        waited = set()
        gathers[0] = pltpu.async_copy(
            table_hbm.at[idx_v.at[0]], rows_v.at[0], gsem)
        for c in range(chunks):
            if c + 1 < chunks:
                if c + 1 >= 3:
                    outs[c - 2].wait()
                    waited.add(c - 2)
                gathers[c + 1] = pltpu.async_copy(
                    table_hbm.at[idx_v.at[c + 1]],
                    rows_v.at[(c + 1) % 3], gsem)
            gathers[c].wait()
            outs[c] = pltpu.async_copy(
                rows_v.at[c % 3], out_hbm.at[pl.ds(base + c * _CH, _CH)],
                osem)
        for c in range(chunks):
            if c not in waited:
                outs[c].wait()

    return gather_kernel(table, idx2)


def kernel(h_est_real, h_est_imag, codebook_real, codebook_imag):
    b, v, s = h_est_real.shape
    k_codes = codebook_real.shape[0]
    d = v * s

    hrt = h_est_real.transpose(2, 1, 0)
    hit = h_est_imag.transpose(2, 1, 0)
    crf = codebook_real.transpose(0, 2, 1).reshape(k_codes, d)
    cif = codebook_imag.transpose(0, 2, 1).reshape(k_codes, d)
    wre = jnp.concatenate([crf, -(crf - cif)], axis=1)
    wim = jnp.concatenate([crf, -(crf + cif)], axis=1)

    d_pad = 2 * _CH
    table = jnp.stack([codebook_real, codebook_imag], axis=-1)
    table = table.reshape(k_codes, 2 * d)
    table = jnp.pad(table, ((0, 0), (0, d_pad - 2 * d)))

    nb = b // _BT
    idx = _tc_scores_argmax(hrt, hit, wre, wim, nb, 0)
    idx2 = idx.reshape(b // _CH, _CH)
    rows = _sc_gather(table, idx2, b, d_pad, 2 * d)
    return rows[:, : 2 * d].reshape(b, v, s, 2)
